# baseline (speedup 1.0000x reference)
import jax
import jax.numpy as jnp
from jax.experimental import pallas as pl
from jax.experimental.pallas import tpu as pltpu

_G = 256
_N = _G * _G
_E = 96
_PACK = 32
_PR = _N // _PACK
_HALO = 8
_BLK = 32768
_BPR = _BLK // _PACK
_STEPS = _N // _BLK


def _dinv_packed(row0, shape):
    gi = row0 + jax.lax.broadcasted_iota(jnp.int32, shape, 0)
    li = jax.lax.broadcasted_iota(jnp.int32, shape, 1)
    h = gi // _HALO
    w = (gi % _HALO) * _PACK + li // 4
    deg = (5.0
           - jnp.where(h == 0, 1.0, 0.0)
           - jnp.where(h == _G - 1, 1.0, 0.0)
           - jnp.where(w == 0, 1.0, 0.0)
           - jnp.where(w == _G - 1, 1.0, 0.0)
           + jnp.where((h == _G - 2) & (w == _G - 2), 1.0, 0.0))
    return jax.lax.rsqrt(deg), (w == 0), (w == _G - 1)


def _kernel(x_ref, wc_ref, wg_ref, b_ref, out_ref, p_ref):
    i = pl.program_id(0)

    @pl.when(i == 0)
    def _pack():
        xr = x_ref[:, :].reshape(_G, 2, 2 * _G)
        xe = xr[:, 0, :]
        xo = xr[:, 1, :]
        ji = jax.lax.broadcasted_iota(jnp.int32, (2 * _G, 4 * _G), 0)
        li = jax.lax.broadcasted_iota(jnp.int32, (2 * _G, 4 * _G), 1)
        tgt = 4 * (ji // 2) + ji % 2
        a_mat = jnp.where(li == tgt, 1.0, 0.0)
        b_mat = jnp.where(li == tgt + 2, 1.0, 0.0)
        p1024 = (jnp.dot(xe, a_mat, preferred_element_type=jnp.float32)
                 + jnp.dot(xo, b_mat, preferred_element_type=jnp.float32))
        hh = jax.lax.broadcasted_iota(jnp.int32, (_G, 4 * _G), 0)
        ww = jax.lax.broadcasted_iota(jnp.int32, (_G, 4 * _G), 1) // 4
        deg = (5.0
               - jnp.where(hh == 0, 1.0, 0.0)
               - jnp.where(hh == _G - 1, 1.0, 0.0)
               - jnp.where(ww == 0, 1.0, 0.0)
               - jnp.where(ww == _G - 1, 1.0, 0.0)
               + jnp.where((hh == _G - 2) & (ww == _G - 2), 1.0, 0.0))
        pz = p1024 * jax.lax.rsqrt(deg)
        p_ref[_HALO:_PR + _HALO, :] = pz.reshape(_PR, 4 * _PACK)
        p_ref[0:_HALO, :] = jnp.zeros((_HALO, 4 * _PACK), jnp.float32)
        p_ref[_PR + _HALO:, :] = jnp.zeros((_HALO, 4 * _PACK), jnp.float32)

    m = jax.lax.dot_general(
        wc_ref[:, :], wg_ref[:, :],
        dimension_numbers=(((0,), (1,)), ((), ())),
        preferred_element_type=jnp.float32)
    wtile = jnp.broadcast_to(m[None, :, :], (_PACK, 4, _E)).reshape(4 * _PACK, _E)

    base = i * _BPR
    pc = p_ref[pl.ds(base + _HALO, _BPR), :]
    pu = p_ref[pl.ds(base, _BPR), :]
    pd = p_ref[pl.ds(base + 2 * _HALO, _BPR), :]

    zrow = jnp.zeros((1, 4 * _PACK), jnp.float32)
    prev = jnp.concatenate([zrow, pc[:-1, :]], axis=0)
    nxt = jnp.concatenate([pc[1:, :], zrow], axis=0)
    zl = jnp.concatenate([prev[:, -4:], pc[:, :-4]], axis=1)
    zr = jnp.concatenate([pc[:, 4:], nxt[:, :4]], axis=1)

    d, w0, w255 = _dinv_packed(base, (_BPR, 4 * _PACK))
    zl = jnp.where(w0, 0.0, zl)
    zr = jnp.where(w255, 0.0, zr)

    s = pc + pu + pd + zl + zr

    src = pc[_BPR - 1:, 4 * _PACK - 4:]
    src_row = jnp.concatenate(
        [jnp.zeros((1, 4 * _PACK - 8), jnp.float32), src,
         jnp.zeros((1, 4), jnp.float32)], axis=1)
    ri0 = jax.lax.broadcasted_iota(jnp.int32, (_BPR, 1), 0)
    stray = jnp.where(ri0 == _BPR - 1 - _HALO, src_row, 0.0)
    s = s + jnp.where(i == _STEPS - 1, 1.0, 0.0) * stray

    sp = s * d

    big = jnp.broadcast_to(sp[:, None, :], (_BPR, _PACK, 4 * _PACK))
    big = big.reshape(_BLK, 4 * _PACK)
    ri = jax.lax.broadcasted_iota(jnp.int32, (_BLK, 4 * _PACK), 0)
    ci = jax.lax.broadcasted_iota(jnp.int32, (_BLK, 4 * _PACK), 1)
    big = jnp.where(ci // 4 == ri % _PACK, big, 0.0)

    u = jnp.dot(big, wtile, preferred_element_type=jnp.float32)
    out_ref[0, :, :] = u + b_ref[0, :]


def kernel(x, Wconv, Wgcn, bgcn):
    xi = x.reshape(2 * _G, 2 * _G)
    wc2 = Wconv.reshape(_E, 4)
    b2 = bgcn.reshape(1, _E)

    out = pl.pallas_call(
        _kernel,
        grid=(_STEPS,),
        in_specs=[
            pl.BlockSpec((2 * _G, 2 * _G), lambda i: (0, 0)),
            pl.BlockSpec((_E, 4), lambda i: (0, 0)),
            pl.BlockSpec((_E, _E), lambda i: (0, 0)),
            pl.BlockSpec((1, _E), lambda i: (0, 0)),
        ],
        out_specs=pl.BlockSpec((1, _BLK, _E), lambda i: (0, i, 0)),
        out_shape=jax.ShapeDtypeStruct((1, _N, _E), jnp.float32),
        scratch_shapes=[pltpu.VMEM((_PR + 2 * _HALO, 4 * _PACK), jnp.float32)],
    )(xi, wc2, Wgcn, b2)
    return out


# speedup vs baseline: 1.0621x; 1.0621x over previous
import jax
import jax.numpy as jnp
from jax.experimental import pallas as pl
from jax.experimental.pallas import tpu as pltpu

_G = 256
_N = _G * _G
_E = 96
_PACK = 32
_PR = _N // _PACK
_HALO = 8
_BLK = 16384
_BPR = _BLK // _PACK
_STEPS = _N // _BLK


def _dinv_packed(row0, shape):
    gi = row0 + jax.lax.broadcasted_iota(jnp.int32, shape, 0)
    li = jax.lax.broadcasted_iota(jnp.int32, shape, 1)
    h = gi // _HALO
    w = (gi % _HALO) * _PACK + li // 4
    deg = (5.0
           - jnp.where(h == 0, 1.0, 0.0)
           - jnp.where(h == _G - 1, 1.0, 0.0)
           - jnp.where(w == 0, 1.0, 0.0)
           - jnp.where(w == _G - 1, 1.0, 0.0)
           + jnp.where((h == _G - 2) & (w == _G - 2), 1.0, 0.0))
    return jax.lax.rsqrt(deg), (w == 0), (w == _G - 1)


def _kernel(x_ref, wc_ref, wg_ref, b_ref, out_ref, p_ref):
    i = pl.program_id(0)

    @pl.when(i == 0)
    def _pack():
        xr = x_ref[:, :].reshape(_G, 2, 2 * _G)
        xe = xr[:, 0, :]
        xo = xr[:, 1, :]
        ji = jax.lax.broadcasted_iota(jnp.int32, (2 * _G, 4 * _G), 0)
        li = jax.lax.broadcasted_iota(jnp.int32, (2 * _G, 4 * _G), 1)
        tgt = 4 * (ji // 2) + ji % 2
        a_mat = jnp.where(li == tgt, 1.0, 0.0)
        b_mat = jnp.where(li == tgt + 2, 1.0, 0.0)
        p1024 = (jnp.dot(xe, a_mat, preferred_element_type=jnp.float32)
                 + jnp.dot(xo, b_mat, preferred_element_type=jnp.float32))
        hh = jax.lax.broadcasted_iota(jnp.int32, (_G, 4 * _G), 0)
        ww = jax.lax.broadcasted_iota(jnp.int32, (_G, 4 * _G), 1) // 4
        deg = (5.0
               - jnp.where(hh == 0, 1.0, 0.0)
               - jnp.where(hh == _G - 1, 1.0, 0.0)
               - jnp.where(ww == 0, 1.0, 0.0)
               - jnp.where(ww == _G - 1, 1.0, 0.0)
               + jnp.where((hh == _G - 2) & (ww == _G - 2), 1.0, 0.0))
        pz = p1024 * jax.lax.rsqrt(deg)
        p_ref[_HALO:_PR + _HALO, :] = pz.reshape(_PR, 4 * _PACK)
        p_ref[0:_HALO, :] = jnp.zeros((_HALO, 4 * _PACK), jnp.float32)
        p_ref[_PR + _HALO:, :] = jnp.zeros((_HALO, 4 * _PACK), jnp.float32)

    m = jax.lax.dot_general(
        wc_ref[:, :], wg_ref[:, :],
        dimension_numbers=(((0,), (1,)), ((), ())),
        preferred_element_type=jnp.float32)
    wtile = jnp.broadcast_to(m[None, :, :], (_PACK, 4, _E)).reshape(4 * _PACK, _E)

    base = i * _BPR
    pc = p_ref[pl.ds(base + _HALO, _BPR), :]
    pu = p_ref[pl.ds(base, _BPR), :]
    pd = p_ref[pl.ds(base + 2 * _HALO, _BPR), :]

    zrow = jnp.zeros((1, 4 * _PACK), jnp.float32)
    prev = jnp.concatenate([zrow, pc[:-1, :]], axis=0)
    nxt = jnp.concatenate([pc[1:, :], zrow], axis=0)
    zl = jnp.concatenate([prev[:, -4:], pc[:, :-4]], axis=1)
    zr = jnp.concatenate([pc[:, 4:], nxt[:, :4]], axis=1)

    d, w0, w255 = _dinv_packed(base, (_BPR, 4 * _PACK))
    zl = jnp.where(w0, 0.0, zl)
    zr = jnp.where(w255, 0.0, zr)

    s = pc + pu + pd + zl + zr

    src = pc[_BPR - 1:, 4 * _PACK - 4:]
    src_row = jnp.concatenate(
        [jnp.zeros((1, 4 * _PACK - 8), jnp.float32), src,
         jnp.zeros((1, 4), jnp.float32)], axis=1)
    ri0 = jax.lax.broadcasted_iota(jnp.int32, (_BPR, 1), 0)
    stray = jnp.where(ri0 == _BPR - 1 - _HALO, src_row, 0.0)
    s = s + jnp.where(i == _STEPS - 1, 1.0, 0.0) * stray

    sp = s * d

    mr = jax.lax.broadcasted_iota(jnp.int32, (_PACK, 4 * _PACK), 0)
    mc = jax.lax.broadcasted_iota(jnp.int32, (_PACK, 4 * _PACK), 1)
    mask = jnp.where(mc // 4 == mr, 1.0, 0.0)
    big = (sp[:, None, :] * mask[None, :, :]).reshape(_BLK, 4 * _PACK)

    u = jnp.dot(big, wtile, preferred_element_type=jnp.float32)
    out_ref[0, :, :] = u + b_ref[0, :]


def kernel(x, Wconv, Wgcn, bgcn):
    xi = x.reshape(2 * _G, 2 * _G)
    wc2 = Wconv.reshape(_E, 4)
    b2 = bgcn.reshape(1, _E)

    out = pl.pallas_call(
        _kernel,
        grid=(_STEPS,),
        in_specs=[
            pl.BlockSpec((2 * _G, 2 * _G), lambda i: (0, 0)),
            pl.BlockSpec((_E, 4), lambda i: (0, 0)),
            pl.BlockSpec((_E, _E), lambda i: (0, 0)),
            pl.BlockSpec((1, _E), lambda i: (0, 0)),
        ],
        out_specs=pl.BlockSpec((1, _BLK, _E), lambda i: (0, i, 0)),
        out_shape=jax.ShapeDtypeStruct((1, _N, _E), jnp.float32),
        scratch_shapes=[pltpu.VMEM((_PR + 2 * _HALO, 4 * _PACK), jnp.float32)],
    )(xi, wc2, Wgcn, b2)
    return out
